# baseline (speedup 1.0000x reference)
import functools

import jax
import jax.numpy as jnp
from jax import lax
from jax.experimental import pallas as pl
from jax.experimental.pallas import tpu as pltpu
from jax.experimental.pallas import tpu_sc as plsc

_L = 16


def _sc_gather_t(tbl1d, idxt, off16, bsz, fc, d, n_rows):
    info = plsc.get_sparse_core_info()
    nc, ns = info.num_cores, info.num_subcores
    nw = nc * ns
    b_per_w = bsz // nw
    n_blk = b_per_w // 128

    mesh = plsc.VectorSubcoreMesh(core_axis_name="c", subcore_axis_name="s")

    @functools.partial(
        pl.kernel,
        mesh=mesh,
        out_type=jax.ShapeDtypeStruct((fc * d * bsz,), jnp.float32),
        scratch_types=[
            pltpu.VMEM((128,), jnp.int32),
            pltpu.VMEM((d * 128,), jnp.int32),
            pltpu.VMEM((d * 128,), jnp.int32),
            pltpu.VMEM((fc * d * 128,), jnp.float32),
            pltpu.VMEM((fc * _L,), jnp.int32),
            pltpu.SemaphoreType.DMA,
        ],
        compiler_params=pltpu.CompilerParams(
            use_tc_tiling_on_sc=False, needs_layout_passes=False
        ),
    )
    def k(tbl_hbm, idx_hbm, off_hbm, out_hbm, base_v, ia_v, ib_v, t_v, off_v,
          sem):
        wid = lax.axis_index("s") * nc + lax.axis_index("c")
        b0w = wid * b_per_w
        pltpu.sync_copy(off_hbm, off_v)

        def per_block(c, carry):
            b0 = b0w + c * 128

            def per_f(f, c2):
                pltpu.sync_copy(idx_hbm.at[pl.ds(f * bsz + b0, 128)], base_v)

                def fill(gd, c3):
                    g = gd // d
                    dd = gd - g * d
                    ia_v[pl.ds(dd * 128 + g * _L, _L)] = (
                        base_v[pl.ds(g * _L, _L)]
                        + off_v[pl.ds(f * _L, _L)]
                        + dd * n_rows
                    )
                    return c3

                lax.fori_loop(0, 8 * d, fill, 0)

                def fire(dd, c3):
                    pltpu.async_copy(
                        tbl_hbm.at[ia_v.at[pl.ds(dd * 128, 128)]],
                        t_v.at[pl.ds(f * d * 128 + dd * 128, 128)],
                        sem,
                    )
                    return c3

                lax.fori_loop(0, d, fire, 0)
                pltpu.make_async_copy(
                    tbl_hbm.at[pl.ds(0, d * 128)],
                    t_v.at[pl.ds(f * d * 128, d * 128)],
                    sem,
                ).wait()
                return c2

            lax.fori_loop(0, fc, per_f, 0)

            def wrow(r, c2):
                pltpu.sync_copy(
                    t_v.at[pl.ds(r * 128, 128)],
                    out_hbm.at[pl.ds(r * bsz + b0, 128)],
                )
                return c2

            lax.fori_loop(0, fc * d, wrow, 0)
            return carry

        lax.fori_loop(0, n_blk, per_block, 0)

    return k(tbl1d, idxt, off16)


def _tc_assemble_t(x_t, w, b, cat_t, block_b=1024):
    f, bsz = x_t.shape
    d = w.shape[1]
    fcd = cat_t.shape[0]

    def body(x_ref, w_ref, b_ref, cat_ref, o_ref):
        o_ref[:f, :, :] = (
            x_ref[...][:, None, :] * w_ref[...][:, :, None]
            + b_ref[...][:, :, None]
        )
        o_ref[f:, :, :] = cat_ref[...].reshape(fcd // d, d, block_b)

    return pl.pallas_call(
        body,
        grid=(bsz // block_b,),
        in_specs=[
            pl.BlockSpec((f, block_b), lambda i: (0, i)),
            pl.BlockSpec((f, d), lambda i: (0, 0)),
            pl.BlockSpec((f, d), lambda i: (0, 0)),
            pl.BlockSpec((fcd, block_b), lambda i: (0, i)),
        ],
        out_specs=pl.BlockSpec((f + fcd // d, d, block_b), lambda i: (0, 0, i)),
        out_shape=jax.ShapeDtypeStruct((f + fcd // d, d, bsz), jnp.float32),
    )(x_t, w, b, cat_t)


def kernel(x_num, x_cat, num_weight, num_bias, cat_table, category_offsets):
    bsz, fc = x_cat.shape
    d = cat_table.shape[1]
    n_rows = cat_table.shape[0]
    idxt = x_cat.T.astype(jnp.int32).reshape(-1)
    off16 = jnp.tile(category_offsets.astype(jnp.int32)[:, None], (1, _L))
    tbl1d = cat_table.T.reshape(-1)
    cat_t = _sc_gather_t(tbl1d, idxt, off16.reshape(-1), bsz, fc, d, n_rows)
    out_t = _tc_assemble_t(x_num.T, num_weight, num_bias,
                           cat_t.reshape(fc * d, bsz))
    return out_t.transpose(2, 0, 1)


# speedup vs baseline: 1.0191x; 1.0191x over previous
import functools

import jax
import jax.numpy as jnp
from jax import lax
from jax.experimental import pallas as pl
from jax.experimental.pallas import tpu as pltpu
from jax.experimental.pallas import tpu_sc as plsc

_L = 16


def _sc_gather_t(tbl1d, idxt, off16, bsz, fc, d, n_rows):
    info = plsc.get_sparse_core_info()
    nc, ns = info.num_cores, info.num_subcores
    nw = nc * ns
    b_per_w = bsz // nw
    n_blk = b_per_w // 128

    mesh = plsc.VectorSubcoreMesh(core_axis_name="c", subcore_axis_name="s")

    @functools.partial(
        pl.kernel,
        mesh=mesh,
        out_type=jax.ShapeDtypeStruct((fc * d * bsz,), jnp.float32),
        scratch_types=[
            pltpu.VMEM((128,), jnp.int32),
            pltpu.VMEM((fc * d * 128,), jnp.int32),
            pltpu.VMEM((fc * d * 128,), jnp.float32),
            pltpu.VMEM((fc * _L,), jnp.int32),
            pltpu.SemaphoreType.DMA,
        ],
        compiler_params=pltpu.CompilerParams(
            use_tc_tiling_on_sc=False, needs_layout_passes=False
        ),
    )
    def k(tbl_hbm, idx_hbm, off_hbm, out_hbm, base_v, ia_v, t_v, off_v, sem):
        wid = lax.axis_index("s") * nc + lax.axis_index("c")
        b0w = wid * b_per_w
        pltpu.sync_copy(off_hbm, off_v)

        def per_block(c, carry):
            b0 = b0w + c * 128

            def per_f(f, c2):
                pltpu.sync_copy(idx_hbm.at[pl.ds(f * bsz + b0, 128)], base_v)

                def fill(gd, c3):
                    g = gd // d
                    dd = gd - g * d
                    ia_v[pl.ds(f * d * 128 + dd * 128 + g * _L, _L)] = (
                        base_v[pl.ds(g * _L, _L)]
                        + off_v[pl.ds(f * _L, _L)]
                        + dd * n_rows
                    )
                    return c3

                lax.fori_loop(0, 8 * d, fill, 0)
                return c2

            lax.fori_loop(0, fc, per_f, 0)
            pltpu.async_copy(tbl_hbm.at[ia_v], t_v, sem).wait()

            def wrow(r, c2):
                pltpu.sync_copy(
                    t_v.at[pl.ds(r * 128, 128)],
                    out_hbm.at[pl.ds(r * bsz + b0, 128)],
                )
                return c2

            lax.fori_loop(0, fc * d, wrow, 0)
            return carry

        lax.fori_loop(0, n_blk, per_block, 0)

    return k(tbl1d, idxt, off16)


def _tc_assemble_t(x_t, w, b, cat_t, block_b=1024):
    f, bsz = x_t.shape
    d = w.shape[1]
    fcd = cat_t.shape[0]

    def body(x_ref, w_ref, b_ref, cat_ref, o_ref):
        o_ref[:f, :, :] = (
            x_ref[...][:, None, :] * w_ref[...][:, :, None]
            + b_ref[...][:, :, None]
        )
        o_ref[f:, :, :] = cat_ref[...].reshape(fcd // d, d, block_b)

    return pl.pallas_call(
        body,
        grid=(bsz // block_b,),
        in_specs=[
            pl.BlockSpec((f, block_b), lambda i: (0, i)),
            pl.BlockSpec((f, d), lambda i: (0, 0)),
            pl.BlockSpec((f, d), lambda i: (0, 0)),
            pl.BlockSpec((fcd, block_b), lambda i: (0, i)),
        ],
        out_specs=pl.BlockSpec((f + fcd // d, d, block_b), lambda i: (0, 0, i)),
        out_shape=jax.ShapeDtypeStruct((f + fcd // d, d, bsz), jnp.float32),
    )(x_t, w, b, cat_t)


def kernel(x_num, x_cat, num_weight, num_bias, cat_table, category_offsets):
    bsz, fc = x_cat.shape
    d = cat_table.shape[1]
    n_rows = cat_table.shape[0]
    idxt = x_cat.T.astype(jnp.int32).reshape(-1)
    off16 = jnp.tile(category_offsets.astype(jnp.int32)[:, None], (1, _L))
    tbl1d = cat_table.T.reshape(-1)
    cat_t = _sc_gather_t(tbl1d, idxt, off16.reshape(-1), bsz, fc, d, n_rows)
    out_t = _tc_assemble_t(x_num.T, num_weight, num_bias,
                           cat_t.reshape(fc * d, bsz))
    return out_t.transpose(2, 0, 1)


# speedup vs baseline: 2.4723x; 2.4259x over previous
import functools

import jax
import jax.numpy as jnp
from jax import lax
from jax.experimental import pallas as pl
from jax.experimental.pallas import tpu as pltpu
from jax.experimental.pallas import tpu_sc as plsc

_LANES = 16


def _sc_gather(table, idx_raw, off_pattern, n_total, d, ch):
    info = plsc.get_sparse_core_info()
    nc, ns = info.num_cores, info.num_subcores
    nw = nc * ns
    n_per_w = n_total // nw
    n_chunks = n_per_w // ch
    assert ch * n_chunks == n_per_w and ch % 8 == 0

    mesh = plsc.VectorSubcoreMesh(core_axis_name="c", subcore_axis_name="s")

    @functools.partial(
        pl.kernel,
        mesh=mesh,
        out_type=jax.ShapeDtypeStruct((n_total, 128), jnp.float32),
        scratch_types=[
            pltpu.VMEM((ch,), jnp.int32),
            pltpu.VMEM((ch, d), jnp.float32),
            pltpu.VMEM((ch,), jnp.int32),
            pltpu.SemaphoreType.DMA,
        ],
        compiler_params=pltpu.CompilerParams(use_tc_tiling_on_sc=False),
    )
    def k(table_hbm, idx_hbm, offp_hbm, out_hbm, idx_v, rows_v, off_v, sem):
        wid = lax.axis_index("s") * nc + lax.axis_index("c")
        base = wid * n_per_w
        pltpu.sync_copy(offp_hbm, off_v)
        for c in range(n_chunks):
            b0 = base + c * ch
            pltpu.sync_copy(idx_hbm.at[pl.ds(b0, ch)], idx_v)

            def add_off(i, carry):
                s = pl.ds(i * _LANES, _LANES)
                idx_v[s] = idx_v[s] + off_v[s]
                return carry

            lax.fori_loop(0, ch // _LANES, add_off, 0)
            pltpu.async_copy(table_hbm.at[idx_v], rows_v, sem).wait()
            pltpu.sync_copy(rows_v, out_hbm.at[pl.ds(b0, ch), pl.ds(0, d)])

    return k(table, idx_raw, off_pattern)


def _tc_assemble(x_num, w, b, cat_stage, fc, block_b=256):
    bsz, f = x_num.shape
    d = w.shape[1]

    def body(x_ref, w_ref, b_ref, cat_ref, o_ref):
        o_ref[:, :f, :] = x_ref[...][:, :, None] * w_ref[...][None] + b_ref[...][None]
        o_ref[:, f:, :] = cat_ref[:, :d].reshape(block_b, fc, d)

    return pl.pallas_call(
        body,
        grid=(bsz // block_b,),
        in_specs=[
            pl.BlockSpec((block_b, f), lambda i: (i, 0)),
            pl.BlockSpec((f, d), lambda i: (0, 0)),
            pl.BlockSpec((f, d), lambda i: (0, 0)),
            pl.BlockSpec((block_b * fc, 128), lambda i: (i, 0)),
        ],
        out_specs=pl.BlockSpec((block_b, f + fc, d), lambda i: (i, 0, 0)),
        out_shape=jax.ShapeDtypeStruct((bsz, f + fc, d), jnp.float32),
    )(x_num, w, b, cat_stage)


def kernel(x_num, x_cat, num_weight, num_bias, cat_table, category_offsets):
    bsz, fc = x_cat.shape
    d = cat_table.shape[1]
    idx_raw = x_cat.astype(jnp.int32).reshape(-1)
    ch = 128 * fc
    off_pattern = jnp.tile(category_offsets.astype(jnp.int32), ch // fc)
    cat_stage = _sc_gather(cat_table, idx_raw, off_pattern, bsz * fc, d, ch)
    return _tc_assemble(x_num, num_weight, num_bias, cat_stage, fc)
